# baseline (speedup 1.0000x reference)
import functools

import jax
import jax.numpy as jnp
from jax import lax
from jax.experimental import pallas as pl
from jax.experimental.pallas import tpu as pltpu
from jax.experimental.pallas import tpu_sc as plsc


def _landmarks_sc(B, V, F, L):
    info = plsc.get_sparse_core_info()
    NC, NS, LANES = info.num_cores, info.num_subcores, info.num_lanes
    NW = NC * NS
    assert B % NW == 0
    b_per_w = B // NW
    P = L * 3
    n_rows = b_per_w * L
    n_out = b_per_w * P
    assert n_rows % LANES == 0 and n_out % LANES == 0
    assert (B * V * 3) % 8 == 0
    IDX_CHUNK = 128
    assert n_rows % IDX_CHUNK == 0
    n_dma = n_rows // IDX_CHUNK
    n_q_chunk = n_rows // LANES
    n_chunk = n_out // LANES

    mesh = plsc.VectorSubcoreMesh(core_axis_name="c", subcore_axis_name="s")

    @functools.partial(
        pl.kernel,
        out_type=jax.ShapeDtypeStruct((B * P,), jnp.float32),
        mesh=mesh,
        scratch_types=(
            [pltpu.VMEM((L,), jnp.int32)]
            + [pltpu.VMEM((L, 8), jnp.int32)]
            + [pltpu.VMEM((P,), jnp.float32)]
            + [pltpu.VMEM((n_rows,), jnp.int32) for _ in range(3)]
            + [pltpu.VMEM((n_rows, 8), jnp.float32) for _ in range(2)]
            + [pltpu.VMEM((n_out,), jnp.float32)]
            + [pltpu.SemaphoreType.DMA]
        ),
        compiler_params=pltpu.CompilerParams(
            use_tc_tiling_on_sc=False, needs_layout_passes=False),
    )
    def k(verts_hbm, faces_hbm, lidx_hbm, bary_hbm, out_hbm,
          lidx_v, f8_v, bary_v, wlo_v, whi_v, off_v, glo_v, ghi_v, out_v, sem):
        wid = lax.axis_index("s") * NC + lax.axis_index("c")
        b0 = wid * b_per_w

        pltpu.sync_copy(lidx_hbm, lidx_v)
        pltpu.sync_copy(bary_hbm, bary_v)
        pltpu.async_copy(faces_hbm.at[lidx_v], f8_v, sem).wait()

        lane = lax.iota(jnp.int32, LANES)
        lvec = jnp.full((LANES,), L, jnp.int32)
        three = jnp.full((LANES,), 3, jnp.int32)
        pvec = jnp.full((LANES,), P, jnp.int32)
        seven = jnp.full((LANES,), 7, jnp.int32)
        eight = jnp.full((LANES,), 8, jnp.int32)
        zero = jnp.full((LANES,), 0, jnp.int32)

        for f in range(3):
            fv = jnp.full((LANES,), f, jnp.int32)

            def build(c, _):
                q = c * LANES + lane
                bl = lax.div(q, lvec)
                l = q - bl * L
                vidx = plsc.load_gather(f8_v, [l, fv])
                s = (b0 + bl) * (V * 3) + vidx * 3
                w = lax.div(s, eight)
                wlo_v[pl.ds(c * LANES, LANES)] = w
                whi_v[pl.ds(c * LANES, LANES)] = lax.min(
                    w + 1, jnp.full((LANES,), B * V * 3 // 8 - 1, jnp.int32))
                off_v[pl.ds(c * LANES, LANES)] = s - w * 8
                return _
            lax.fori_loop(0, n_q_chunk, build, None)

            handles = []
            for j in range(n_dma):
                sl = pl.ds(j * IDX_CHUNK, IDX_CHUNK)
                handles.append(pltpu.async_copy(
                    verts_hbm.at[wlo_v.at[sl]], glo_v.at[sl, :], sem))
                handles.append(pltpu.async_copy(
                    verts_hbm.at[whi_v.at[sl]], ghi_v.at[sl, :], sem))
            for h in handles:
                h.wait()

            def combine(c, _):
                base = c * LANES
                p = base + lane
                q = lax.div(p, three)
                kk = p - q * 3
                r = p - lax.div(p, pvec) * P
                wcoef = plsc.load_gather(bary_v, [r - lax.rem(r, three) + f])
                off_q = plsc.load_gather(off_v, [q])
                co = off_q + kk
                a = plsc.load_gather(glo_v, [q, lax.min(co, seven)])
                bb = plsc.load_gather(ghi_v, [q, lax.max(co - 8, zero)])
                val = jnp.where(co < eight, a, bb)
                contrib = wcoef * val
                if f == 0:
                    out_v[pl.ds(base, LANES)] = contrib
                else:
                    out_v[pl.ds(base, LANES)] = out_v[pl.ds(base, LANES)] + contrib
                return _
            lax.fori_loop(0, n_chunk, combine, None)

        pltpu.sync_copy(out_v, out_hbm.at[pl.ds(wid * n_out, n_out)])

    return k


def kernel(vertices, full_pose, faces, lmk_faces_idx, lmk_bary_coords):
    del full_pose
    B, V = vertices.shape[:2]
    F = faces.shape[0]
    L = lmk_faces_idx.shape[0]
    verts_w8 = vertices.reshape(B * V * 3 // 8, 8)
    faces_p8 = jnp.pad(faces.astype(jnp.int32), ((0, 0), (0, 5)))
    k = _landmarks_sc(B, V, F, L)
    out = k(verts_w8,
            faces_p8,
            lmk_faces_idx.astype(jnp.int32),
            lmk_bary_coords.astype(jnp.float32).reshape(L * 3))
    return out.reshape(B, L, 3)
